# baseline (speedup 1.0000x reference)
import functools

import jax
import jax.numpy as jnp
from jax import lax
from jax.experimental import pallas as pl
from jax.experimental.pallas import tpu as pltpu
from jax.experimental.pallas import tpu_sc as plsc

D = 256
SUB = 128
SQRT_N = 512
K = 8
GSIZE = 16
NCHUNK = SQRT_N // GSIZE
LANES = 16
NEG_INF = float("-inf")

BLOCK_R = 512
SC_BATCH = 32


def _tc_body(q_ref, ck1_ref, ck2_ref, w_ref, b_ref,
             s1_ref, s2_ref, cm1_ref, cm2_ref, aux_ref, ps_ref, *, grid):
    i = pl.program_id(0)

    @pl.when(i == 0)
    def _init():
        ps_ref[...] = jnp.zeros_like(ps_ref)

    q = q_ref[...]
    mu = jnp.mean(q, axis=-1, keepdims=True)
    var = jnp.mean((q - mu) ** 2, axis=-1, keepdims=True)
    qn = (q - mu) / jnp.sqrt(var + 1e-5) * w_ref[...] + b_ref[...]

    s1 = jnp.dot(qn[:, :SUB], ck1_ref[...], preferred_element_type=jnp.float32)
    s2 = jnp.dot(qn[:, SUB:], ck2_ref[...], preferred_element_type=jnp.float32)
    s1_ref[...] = s1
    s2_ref[...] = s2

    r = s1.shape[0]
    cm1_ref[...] = jnp.max(s1.reshape(r, NCHUNK, GSIZE), axis=-1)
    cm2_ref[...] = jnp.max(s2.reshape(r, NCHUNK, GSIZE), axis=-1)

    e1 = jnp.exp(s1)
    e2 = jnp.exp(s2)
    p1 = e1 / jnp.sum(e1, axis=-1, keepdims=True)
    p2 = e2 / jnp.sum(e2, axis=-1, keepdims=True)
    ps_ref[0:1, :] = ps_ref[0:1, :] + jnp.sum(p1, axis=0, keepdims=True)
    ps_ref[1:2, :] = ps_ref[1:2, :] + jnp.sum(p2, axis=0, keepdims=True)

    @pl.when(i == grid - 1)
    def _fin():
        n_rows = grid * r
        ps = ps_ref[...] * (1.0 / n_rows)
        aux_ref[...] = jnp.sum(ps * ps).reshape(1, 1) * SQRT_N


def _i16():
    return lax.iota(jnp.int32, LANES)


def _gather16(x, idx):
    return lax.gather(
        x, idx.reshape(LANES, 1),
        lax.GatherDimensionNumbers(offset_dims=(), collapsed_slice_dims=(0,),
                                   start_index_map=(0,)),
        (1,), mode=lax.GatherScatterMode.PROMISE_IN_BOUNDS)


def _merge_desc(xk, xv, yk, yv):
    ryk = lax.rev(yk, (0,))
    ryv = lax.rev(yv, (0,))
    m = xk >= ryk
    return plsc.sort_key_val(jnp.where(m, xk, ryk), jnp.where(m, xv, ryv),
                             descending=True)


def _sc_body(rows_w, batch, nw,
             s1_hbm, s2_hbm, cm1_hbm, cm2_hbm, outi_hbm, outf_hbm,
             cm1_v, cm2_v, ids1_v, ids2_v, cand1_v, cand2_v,
             outi_v, outf_v, sem1, sem2):
    wid = lax.axis_index("s") * 2 + lax.axis_index("c")
    row0 = wid * rows_w
    nb = rows_w // batch
    iota = _i16()
    hi32 = jnp.where(iota < K, 0, NCHUNK)
    shift8 = jnp.maximum(iota - K, 0)
    a8 = (iota >= 8).astype(jnp.int32)
    a12 = (iota >= 12).astype(jnp.int32)
    a14 = (iota >= 14).astype(jnp.int32)
    pi_a = a8 + a12 + a14
    pj_a = iota - (8 * a8 + 4 * a12 + 2 * a14)
    ca = pi_a * K + pj_a
    pi_b = jnp.where(iota < 4, iota + 4, 0)
    pj_b = iota * 0
    cb = pi_b * K

    def top8_chunks(cm_v, off):
        ak = cm_v[pl.ds(off, LANES)]
        bk = cm_v[pl.ds(off + LANES, LANES)]
        ask, asv = plsc.sort_key_val(ak, iota, descending=True)
        bsk, bsv = plsc.sort_key_val(bk, iota + LANES, descending=True)
        _, mv = _merge_desc(ask, asv, bsk, bsv)
        return mv

    def batch_body(bb, carry):
        rb = row0 + bb * batch

        pltpu.sync_copy(cm1_hbm.at[pl.ds(rb * NCHUNK, batch * NCHUNK)], cm1_v)
        pltpu.sync_copy(cm2_hbm.at[pl.ds(rb * NCHUNK, batch * NCHUNK)], cm2_v)

        def pair_body(pp, c):
            r = 2 * pp
            for cm_v, ids_v, cand_v, s_hbm, sem in (
                (cm1_v, ids1_v, cand1_v, s1_hbm, sem1),
                (cm2_v, ids2_v, cand2_v, s2_hbm, sem2),
            ):
                mv0 = top8_chunks(cm_v, r * NCHUNK)
                mv1 = top8_chunks(cm_v, (r + 1) * NCHUNK)
                ids16 = jnp.where(iota < K, mv0, _gather16(mv1, shift8))
                ids_v[pl.ds(r * K, LANES)] = ids16
                fidx = ids16 + (rb + r) * NCHUNK + hi32
                pltpu.async_copy(s_hbm.at[fidx],
                                 cand_v.at[pl.ds(r * K, LANES)], sem)
            return c
        lax.fori_loop(0, batch // 2, pair_body, 0)

        pltpu.make_async_copy(s1_hbm.at[pl.ds(0, batch * K)], cand1_v, sem1).wait()
        pltpu.make_async_copy(s2_hbm.at[pl.ds(0, batch * K)], cand2_v, sem2).wait()

        def row_body(r, c):
            tops = []
            for ids_v, cand_v in ((ids1_v, cand1_v), (ids2_v, cand2_v)):
                idsvec = ids_v[pl.ds(r * K, LANES)]
                runs = []
                for j in range(K):
                    p = r * K + j
                    v = cand_v[p, :]
                    ge = _gather16(idsvec, iota * 0 + j) * GSIZE + iota
                    runs.append(plsc.sort_key_val(v, ge, descending=True))
                while len(runs) > 1:
                    runs = [_merge_desc(*runs[t], *runs[t + 1])
                            for t in range(0, len(runs), 2)]
                tops.append(runs[0])
            (s1k, s1v), (s2k, s2v) = tops

            ka = _gather16(s1k, pi_a) + _gather16(s2k, pj_a)
            kb = _gather16(s1k, pi_b) + _gather16(s2k, pj_b)
            kb = jnp.where(iota < 4, kb, NEG_INF)
            ask, acs = plsc.sort_key_val(ka, ca, descending=True)
            fk, fc = _merge_desc(ask, acs, kb, cb)

            fi = lax.shift_right_logical(fc, 3)
            fj = jnp.bitwise_and(fc, 7)
            rr = _gather16(s1v, fi)
            rc = _gather16(s2v, fj)
            gi = rr * SQRT_N + rc
            outi_v[pl.ds(r * K, LANES)] = gi
            outf_v[pl.ds(r * K, LANES)] = fk
            return c
        lax.fori_loop(0, batch, row_body, 0)

        pltpu.sync_copy(outi_v.at[pl.ds(0, batch * K)],
                        outi_hbm.at[pl.ds(rb * K, batch * K)])
        pltpu.sync_copy(outf_v.at[pl.ds(0, batch * K)],
                        outf_hbm.at[pl.ds(rb * K, batch * K)])
        return carry

    lax.fori_loop(0, nb, batch_body, 0)


@jax.jit
def _run(query, c_keys, c_prime_keys, ln_weight, ln_bias):
    b, s, h, d = query.shape
    n = b * s * h
    q = query.reshape(n, d)
    ck1 = c_keys.T
    ck2 = c_prime_keys.T
    w = ln_weight.reshape(1, d)
    bias = ln_bias.reshape(1, d)

    grid = n // BLOCK_R
    s1, s2, cm1, cm2, aux = pl.pallas_call(
        functools.partial(_tc_body, grid=grid),
        grid=(grid,),
        in_specs=[
            pl.BlockSpec((BLOCK_R, d), lambda i: (i, 0)),
            pl.BlockSpec((SUB, SQRT_N), lambda i: (0, 0)),
            pl.BlockSpec((SUB, SQRT_N), lambda i: (0, 0)),
            pl.BlockSpec((1, d), lambda i: (0, 0)),
            pl.BlockSpec((1, d), lambda i: (0, 0)),
        ],
        out_specs=[
            pl.BlockSpec((BLOCK_R, SQRT_N), lambda i: (i, 0)),
            pl.BlockSpec((BLOCK_R, SQRT_N), lambda i: (i, 0)),
            pl.BlockSpec((BLOCK_R, NCHUNK), lambda i: (i, 0)),
            pl.BlockSpec((BLOCK_R, NCHUNK), lambda i: (i, 0)),
            pl.BlockSpec((1, 1), lambda i: (0, 0)),
        ],
        out_shape=[
            jax.ShapeDtypeStruct((n, SQRT_N), jnp.float32),
            jax.ShapeDtypeStruct((n, SQRT_N), jnp.float32),
            jax.ShapeDtypeStruct((n, NCHUNK), jnp.float32),
            jax.ShapeDtypeStruct((n, NCHUNK), jnp.float32),
            jax.ShapeDtypeStruct((1, 1), jnp.float32),
        ],
        scratch_shapes=[pltpu.VMEM((2, SQRT_N), jnp.float32)],
    )(q, ck1, ck2, w, bias)

    info = plsc.get_sparse_core_info()
    nw = info.num_cores * info.num_subcores
    rows_w = n // nw
    batch = SC_BATCH
    mesh = plsc.VectorSubcoreMesh(core_axis_name="c", subcore_axis_name="s")

    sc_fn = pl.kernel(
        functools.partial(_sc_body, rows_w, batch, nw),
        out_type=[
            jax.ShapeDtypeStruct((n * K,), jnp.int32),
            jax.ShapeDtypeStruct((n * K,), jnp.float32),
        ],
        mesh=mesh,
        compiler_params=pltpu.CompilerParams(use_tc_tiling_on_sc=False,
                                             needs_layout_passes=False),
        scratch_types=[
            pltpu.VMEM((batch * NCHUNK,), jnp.float32),
            pltpu.VMEM((batch * NCHUNK,), jnp.float32),
            pltpu.VMEM((batch * K + K,), jnp.int32),
            pltpu.VMEM((batch * K + K,), jnp.int32),
            pltpu.VMEM((batch * K, GSIZE), jnp.float32),
            pltpu.VMEM((batch * K, GSIZE), jnp.float32),
            pltpu.VMEM((batch * K + K,), jnp.int32),
            pltpu.VMEM((batch * K + K,), jnp.float32),
            pltpu.SemaphoreType.DMA,
            pltpu.SemaphoreType.DMA,
        ],
    )
    gidx, fsc = sc_fn(s1.reshape(n * NCHUNK, GSIZE),
                      s2.reshape(n * NCHUNK, GSIZE),
                      cm1.reshape(n * NCHUNK), cm2.reshape(n * NCHUNK))

    return (gidx.reshape(b, s, h, K), fsc.reshape(b, s, h, K),
            aux.reshape(()))


def kernel(query, c_keys, c_prime_keys, ln_weight, ln_bias):
    return _run(query, c_keys, c_prime_keys, ln_weight, ln_bias)


# speedup vs baseline: 1.4715x; 1.4715x over previous
import functools

import jax
import jax.numpy as jnp
from jax import lax
from jax.experimental import pallas as pl
from jax.experimental.pallas import tpu as pltpu
from jax.experimental.pallas import tpu_sc as plsc

D = 256
SUB = 128
SQRT_N = 512
K = 8
GSIZE = 16
NCHUNK = SQRT_N // GSIZE
LANES = 16
NEG_INF = float("-inf")

BLOCK_R = 512
SC_BATCH = 32


def _tc_body(q_ref, ck1_ref, ck2_ref, w_ref, b_ref,
             s1_ref, s2_ref, cm1_ref, cm2_ref, aux_ref, ps_ref, *, grid):
    i = pl.program_id(0)

    @pl.when(i == 0)
    def _init():
        ps_ref[...] = jnp.zeros_like(ps_ref)

    q = q_ref[...]
    mu = jnp.mean(q, axis=-1, keepdims=True)
    var = jnp.mean((q - mu) ** 2, axis=-1, keepdims=True)
    qn = (q - mu) / jnp.sqrt(var + 1e-5) * w_ref[...] + b_ref[...]

    s1 = jnp.dot(qn[:, :SUB], ck1_ref[...], preferred_element_type=jnp.float32)
    s2 = jnp.dot(qn[:, SUB:], ck2_ref[...], preferred_element_type=jnp.float32)
    s1_ref[...] = s1
    s2_ref[...] = s2

    sel = jnp.equal(
        lax.broadcasted_iota(jnp.int32, (SQRT_N, NCHUNK), 0),
        GSIZE * lax.broadcasted_iota(jnp.int32, (SQRT_N, NCHUNK), 1),
    ).astype(jnp.float32)

    def chunk_max(s):
        x = s
        for sh in (1, 2, 4, 8):
            x = jnp.maximum(x, pltpu.roll(x, SQRT_N - sh, 1))
        return jnp.dot(x, sel, preferred_element_type=jnp.float32,
                       precision=lax.Precision.HIGHEST)

    cm1_ref[...] = chunk_max(s1)
    cm2_ref[...] = chunk_max(s2)

    e1 = jnp.exp(s1)
    e2 = jnp.exp(s2)
    p1 = e1 / jnp.sum(e1, axis=-1, keepdims=True)
    p2 = e2 / jnp.sum(e2, axis=-1, keepdims=True)
    ps_ref[0:1, :] = ps_ref[0:1, :] + jnp.sum(p1, axis=0, keepdims=True)
    ps_ref[1:2, :] = ps_ref[1:2, :] + jnp.sum(p2, axis=0, keepdims=True)

    @pl.when(i == grid - 1)
    def _fin():
        n_rows = grid * BLOCK_R
        ps = ps_ref[...] * (1.0 / n_rows)
        aux_ref[...] = jnp.sum(ps * ps).reshape(1, 1) * SQRT_N


def _i16():
    return lax.iota(jnp.int32, LANES)


def _gather16(x, idx):
    return lax.gather(
        x, idx.reshape(LANES, 1),
        lax.GatherDimensionNumbers(offset_dims=(), collapsed_slice_dims=(0,),
                                   start_index_map=(0,)),
        (1,), mode=lax.GatherScatterMode.PROMISE_IN_BOUNDS)


def _merge_desc(xk, xv, yk, yv):
    ryk = lax.rev(yk, (0,))
    ryv = lax.rev(yv, (0,))
    m = xk >= ryk
    return plsc.sort_key_val(jnp.where(m, xk, ryk), jnp.where(m, xv, ryv),
                             descending=True)


def _sc_body(rows_w, batch, nw,
             s1_hbm, s2_hbm, cm1_hbm, cm2_hbm, outi_hbm, outf_hbm,
             cm1_v, cm2_v, ids1_v, ids2_v, cand1_v, cand2_v,
             outi_v, outf_v, sem1, sem2):
    wid = lax.axis_index("s") * 2 + lax.axis_index("c")
    row0 = wid * rows_w
    nb = rows_w // batch
    iota = _i16()
    hi32 = jnp.where(iota < K, 0, NCHUNK)
    shift8 = jnp.maximum(iota - K, 0)
    a8 = (iota >= 8).astype(jnp.int32)
    a12 = (iota >= 12).astype(jnp.int32)
    a14 = (iota >= 14).astype(jnp.int32)
    pi_a = a8 + a12 + a14
    pj_a = iota - (8 * a8 + 4 * a12 + 2 * a14)
    ca = pi_a * K + pj_a
    pi_b = jnp.where(iota < 4, iota + 4, 0)
    pj_b = iota * 0
    cb = pi_b * K

    def top8_chunks(cm_v, off):
        ak = cm_v[pl.ds(off, LANES)]
        bk = cm_v[pl.ds(off + LANES, LANES)]
        ask, asv = plsc.sort_key_val(ak, iota, descending=True)
        bsk, bsv = plsc.sort_key_val(bk, iota + LANES, descending=True)
        _, mv = _merge_desc(ask, asv, bsk, bsv)
        return mv

    def batch_body(bb, carry):
        rb = row0 + bb * batch

        pltpu.sync_copy(cm1_hbm.at[pl.ds(rb * NCHUNK, batch * NCHUNK)], cm1_v)
        pltpu.sync_copy(cm2_hbm.at[pl.ds(rb * NCHUNK, batch * NCHUNK)], cm2_v)

        def pair_body(pp, c):
            r = 2 * pp
            for cm_v, ids_v, cand_v, s_hbm, sem in (
                (cm1_v, ids1_v, cand1_v, s1_hbm, sem1),
                (cm2_v, ids2_v, cand2_v, s2_hbm, sem2),
            ):
                mv0 = top8_chunks(cm_v, r * NCHUNK)
                mv1 = top8_chunks(cm_v, (r + 1) * NCHUNK)
                ids16 = jnp.where(iota < K, mv0, _gather16(mv1, shift8))
                ids_v[pl.ds(r * K, LANES)] = ids16
                fidx = ids16 + (rb + r) * NCHUNK + hi32
                pltpu.async_copy(s_hbm.at[fidx],
                                 cand_v.at[pl.ds(r * K, LANES)], sem)
            return c
        lax.fori_loop(0, batch // 2, pair_body, 0)

        pltpu.make_async_copy(s1_hbm.at[pl.ds(0, batch * K)], cand1_v, sem1).wait()
        pltpu.make_async_copy(s2_hbm.at[pl.ds(0, batch * K)], cand2_v, sem2).wait()

        def row_body(r, c):
            tops = []
            for ids_v, cand_v in ((ids1_v, cand1_v), (ids2_v, cand2_v)):
                idsvec = ids_v[pl.ds(r * K, LANES)]
                runs = []
                for j in range(K):
                    p = r * K + j
                    v = cand_v[p, :]
                    ge = _gather16(idsvec, iota * 0 + j) * GSIZE + iota
                    runs.append(plsc.sort_key_val(v, ge, descending=True))
                while len(runs) > 1:
                    runs = [_merge_desc(*runs[t], *runs[t + 1])
                            for t in range(0, len(runs), 2)]
                tops.append(runs[0])
            (s1k, s1v), (s2k, s2v) = tops

            ka = _gather16(s1k, pi_a) + _gather16(s2k, pj_a)
            kb = _gather16(s1k, pi_b) + _gather16(s2k, pj_b)
            kb = jnp.where(iota < 4, kb, NEG_INF)
            ask, acs = plsc.sort_key_val(ka, ca, descending=True)
            fk, fc = _merge_desc(ask, acs, kb, cb)

            fi = lax.shift_right_logical(fc, 3)
            fj = jnp.bitwise_and(fc, 7)
            rr = _gather16(s1v, fi)
            rc = _gather16(s2v, fj)
            gi = rr * SQRT_N + rc
            outi_v[pl.ds(r * K, LANES)] = gi
            outf_v[pl.ds(r * K, LANES)] = fk
            return c
        lax.fori_loop(0, batch, row_body, 0)

        pltpu.sync_copy(outi_v.at[pl.ds(0, batch * K)],
                        outi_hbm.at[pl.ds(rb * K, batch * K)])
        pltpu.sync_copy(outf_v.at[pl.ds(0, batch * K)],
                        outf_hbm.at[pl.ds(rb * K, batch * K)])
        return carry

    lax.fori_loop(0, nb, batch_body, 0)


@jax.jit
def _run(query, c_keys, c_prime_keys, ln_weight, ln_bias):
    b, s, h, d = query.shape
    n = b * s * h
    q = query.reshape(n, d)
    ck1 = c_keys.T
    ck2 = c_prime_keys.T
    w = ln_weight.reshape(1, d)
    bias = ln_bias.reshape(1, d)

    grid = n // BLOCK_R
    s1, s2, cm1, cm2, aux = pl.pallas_call(
        functools.partial(_tc_body, grid=grid),
        grid=(grid,),
        in_specs=[
            pl.BlockSpec((BLOCK_R, d), lambda i: (i, 0)),
            pl.BlockSpec((SUB, SQRT_N), lambda i: (0, 0)),
            pl.BlockSpec((SUB, SQRT_N), lambda i: (0, 0)),
            pl.BlockSpec((1, d), lambda i: (0, 0)),
            pl.BlockSpec((1, d), lambda i: (0, 0)),
        ],
        out_specs=[
            pl.BlockSpec((BLOCK_R, SQRT_N), lambda i: (i, 0)),
            pl.BlockSpec((BLOCK_R, SQRT_N), lambda i: (i, 0)),
            pl.BlockSpec((BLOCK_R, NCHUNK), lambda i: (i, 0)),
            pl.BlockSpec((BLOCK_R, NCHUNK), lambda i: (i, 0)),
            pl.BlockSpec((1, 1), lambda i: (0, 0)),
        ],
        out_shape=[
            jax.ShapeDtypeStruct((n, SQRT_N), jnp.float32),
            jax.ShapeDtypeStruct((n, SQRT_N), jnp.float32),
            jax.ShapeDtypeStruct((n, NCHUNK), jnp.float32),
            jax.ShapeDtypeStruct((n, NCHUNK), jnp.float32),
            jax.ShapeDtypeStruct((1, 1), jnp.float32),
        ],
        scratch_shapes=[pltpu.VMEM((2, SQRT_N), jnp.float32)],
    )(q, ck1, ck2, w, bias)

    info = plsc.get_sparse_core_info()
    nw = info.num_cores * info.num_subcores
    rows_w = n // nw
    batch = SC_BATCH
    mesh = plsc.VectorSubcoreMesh(core_axis_name="c", subcore_axis_name="s")

    sc_fn = pl.kernel(
        functools.partial(_sc_body, rows_w, batch, nw),
        out_type=[
            jax.ShapeDtypeStruct((n * K,), jnp.int32),
            jax.ShapeDtypeStruct((n * K,), jnp.float32),
        ],
        mesh=mesh,
        compiler_params=pltpu.CompilerParams(use_tc_tiling_on_sc=False,
                                             needs_layout_passes=False),
        scratch_types=[
            pltpu.VMEM((batch * NCHUNK,), jnp.float32),
            pltpu.VMEM((batch * NCHUNK,), jnp.float32),
            pltpu.VMEM((batch * K + K,), jnp.int32),
            pltpu.VMEM((batch * K + K,), jnp.int32),
            pltpu.VMEM((batch * K, GSIZE), jnp.float32),
            pltpu.VMEM((batch * K, GSIZE), jnp.float32),
            pltpu.VMEM((batch * K + K,), jnp.int32),
            pltpu.VMEM((batch * K + K,), jnp.float32),
            pltpu.SemaphoreType.DMA,
            pltpu.SemaphoreType.DMA,
        ],
    )
    gidx, fsc = sc_fn(s1.reshape(n * NCHUNK, GSIZE),
                      s2.reshape(n * NCHUNK, GSIZE),
                      cm1.reshape(n * NCHUNK), cm2.reshape(n * NCHUNK))

    return (gidx.reshape(b, s, h, K), fsc.reshape(b, s, h, K),
            aux.reshape(()))


def kernel(query, c_keys, c_prime_keys, ln_weight, ln_bias):
    return _run(query, c_keys, c_prime_keys, ln_weight, ln_bias)


# speedup vs baseline: 1.7046x; 1.1584x over previous
import functools

import jax
import jax.numpy as jnp
from jax import lax
from jax.experimental import pallas as pl
from jax.experimental.pallas import tpu as pltpu
from jax.experimental.pallas import tpu_sc as plsc

D = 256
SUB = 128
SQRT_N = 512
K = 8
GSIZE = 16
NCHUNK = SQRT_N // GSIZE
LANES = 16
NEG_INF = float("-inf")

BLOCK_R = 512
SC_BATCH = 32


def _tc_body(q_ref, ck1_ref, ck2_ref, w_ref, b_ref,
             s1_ref, s2_ref, cm1_ref, cm2_ref, aux_ref, ps_ref, *, grid):
    i = pl.program_id(0)

    @pl.when(i == 0)
    def _init():
        ps_ref[...] = jnp.zeros_like(ps_ref)

    q = q_ref[...]
    mu = jnp.mean(q, axis=-1, keepdims=True)
    var = jnp.mean((q - mu) ** 2, axis=-1, keepdims=True)
    qn = (q - mu) / jnp.sqrt(var + 1e-5) * w_ref[...] + b_ref[...]

    s1 = jnp.dot(qn[:, :SUB], ck1_ref[...], preferred_element_type=jnp.float32)
    s2 = jnp.dot(qn[:, SUB:], ck2_ref[...], preferred_element_type=jnp.float32)
    for bb in range(4):
        s1_ref[bb, :, :] = s1[:, bb * 128:(bb + 1) * 128]
        s2_ref[bb, :, :] = s2[:, bb * 128:(bb + 1) * 128]

    sel = jnp.equal(
        lax.broadcasted_iota(jnp.int32, (SQRT_N, NCHUNK), 0),
        GSIZE * lax.broadcasted_iota(jnp.int32, (SQRT_N, NCHUNK), 1),
    ).astype(jnp.float32)

    def chunk_max(s):
        x = s
        for sh in (1, 2, 4, 8):
            x = jnp.maximum(x, pltpu.roll(x, SQRT_N - sh, 1))
        return jnp.dot(x, sel, preferred_element_type=jnp.float32,
                       precision=lax.Precision.HIGHEST)

    cm1_ref[...] = chunk_max(s1)
    cm2_ref[...] = chunk_max(s2)

    e1 = jnp.exp(s1)
    e2 = jnp.exp(s2)
    p1 = e1 / jnp.sum(e1, axis=-1, keepdims=True)
    p2 = e2 / jnp.sum(e2, axis=-1, keepdims=True)
    ps_ref[0:1, :] = ps_ref[0:1, :] + jnp.sum(p1, axis=0, keepdims=True)
    ps_ref[1:2, :] = ps_ref[1:2, :] + jnp.sum(p2, axis=0, keepdims=True)

    @pl.when(i == grid - 1)
    def _fin():
        n_rows = grid * BLOCK_R
        ps = ps_ref[...] * (1.0 / n_rows)
        aux_ref[...] = jnp.sum(ps * ps).reshape(1, 1) * SQRT_N


def _i16():
    return lax.iota(jnp.int32, LANES)


def _gather16(x, idx):
    return lax.gather(
        x, idx.reshape(LANES, 1),
        lax.GatherDimensionNumbers(offset_dims=(), collapsed_slice_dims=(0,),
                                   start_index_map=(0,)),
        (1,), mode=lax.GatherScatterMode.PROMISE_IN_BOUNDS)


def _merge_desc(xk, xv, yk, yv):
    ryk = lax.rev(yk, (0,))
    ryv = lax.rev(yv, (0,))
    m = xk >= ryk
    return plsc.sort_key_val(jnp.where(m, xk, ryk), jnp.where(m, xv, ryv),
                             descending=True)


def _sc_body(rows_w, batch, nw,
             s1_hbm, s2_hbm, cm1_hbm, cm2_hbm, outi_hbm, outf_hbm,
             cm1_v, cm2_v, ids1_v, ids2_v, cand1_v, cand2_v,
             outi_v, outf_v, sem1, sem2):
    wid = lax.axis_index("s") * 2 + lax.axis_index("c")
    row0 = wid * rows_w
    nb = rows_w // batch
    iota = _i16()
    n8 = rows_w * nw * 8
    shift8 = jnp.maximum(iota - K, 0)
    a8 = (iota >= 8).astype(jnp.int32)
    a12 = (iota >= 12).astype(jnp.int32)
    a14 = (iota >= 14).astype(jnp.int32)
    pi_a = a8 + a12 + a14
    pj_a = iota - (8 * a8 + 4 * a12 + 2 * a14)
    ca = pi_a * K + pj_a
    pi_b = jnp.where(iota < 4, iota + 4, 0)
    pj_b = iota * 0
    cb = pi_b * K

    def top8_chunks(cm_v, off):
        ak = cm_v[pl.ds(off, LANES)]
        bk = cm_v[pl.ds(off + LANES, LANES)]
        ask, asv = plsc.sort_key_val(ak, iota, descending=True)
        bsk, bsv = plsc.sort_key_val(bk, iota + LANES, descending=True)
        _, mv = _merge_desc(ask, asv, bsk, bsv)
        return mv

    def batch_body(bb, carry):
        rb = row0 + bb * batch

        pltpu.sync_copy(cm1_hbm.at[pl.ds(rb * NCHUNK, batch * NCHUNK)], cm1_v)
        pltpu.sync_copy(cm2_hbm.at[pl.ds(rb * NCHUNK, batch * NCHUNK)], cm2_v)

        def pair_body(pp, c):
            r = 2 * pp
            for cm_v, ids_v, cand_v, s_hbm, sem in (
                (cm1_v, ids1_v, cand1_v, s1_hbm, sem1),
                (cm2_v, ids2_v, cand2_v, s2_hbm, sem2),
            ):
                mv0 = top8_chunks(cm_v, r * NCHUNK)
                mv1 = top8_chunks(cm_v, (r + 1) * NCHUNK)
                ids16 = jnp.where(iota < K, mv0, _gather16(mv1, shift8))
                ids_v[pl.ds(r * K, LANES)] = ids16
                fidx = (lax.shift_right_logical(ids16, 3) * n8
                        + ((rb + r) + a8) * 8 + jnp.bitwise_and(ids16, 7))
                pltpu.async_copy(s_hbm.at[fidx],
                                 cand_v.at[pl.ds(r * K, LANES)], sem)
            return c
        lax.fori_loop(0, batch // 2, pair_body, 0)

        pltpu.make_async_copy(s1_hbm.at[pl.ds(0, batch * K)], cand1_v, sem1).wait()
        pltpu.make_async_copy(s2_hbm.at[pl.ds(0, batch * K)], cand2_v, sem2).wait()

        def row_body(r, c):
            tops = []
            for ids_v, cand_v in ((ids1_v, cand1_v), (ids2_v, cand2_v)):
                idsvec = ids_v[pl.ds(r * K, LANES)]
                runs = []
                for j in range(K):
                    p = r * K + j
                    v = cand_v[p, :]
                    ge = _gather16(idsvec, iota * 0 + j) * GSIZE + iota
                    runs.append(plsc.sort_key_val(v, ge, descending=True))
                while len(runs) > 1:
                    runs = [_merge_desc(*runs[t], *runs[t + 1])
                            for t in range(0, len(runs), 2)]
                tops.append(runs[0])
            (s1k, s1v), (s2k, s2v) = tops

            ka = _gather16(s1k, pi_a) + _gather16(s2k, pj_a)
            kb = _gather16(s1k, pi_b) + _gather16(s2k, pj_b)
            kb = jnp.where(iota < 4, kb, NEG_INF)
            ask, acs = plsc.sort_key_val(ka, ca, descending=True)
            fk, fc = _merge_desc(ask, acs, kb, cb)

            fi = lax.shift_right_logical(fc, 3)
            fj = jnp.bitwise_and(fc, 7)
            rr = _gather16(s1v, fi)
            rc = _gather16(s2v, fj)
            gi = rr * SQRT_N + rc
            outi_v[pl.ds(r * K, LANES)] = gi
            outf_v[pl.ds(r * K, LANES)] = fk
            return c
        lax.fori_loop(0, batch, row_body, 0)

        pltpu.sync_copy(outi_v.at[pl.ds(0, batch * K)],
                        outi_hbm.at[pl.ds(rb * K, batch * K)])
        pltpu.sync_copy(outf_v.at[pl.ds(0, batch * K)],
                        outf_hbm.at[pl.ds(rb * K, batch * K)])
        return carry

    lax.fori_loop(0, nb, batch_body, 0)


@jax.jit
def _run(query, c_keys, c_prime_keys, ln_weight, ln_bias):
    b, s, h, d = query.shape
    n = b * s * h
    q = query.reshape(n, d)
    ck1 = c_keys.T
    ck2 = c_prime_keys.T
    w = ln_weight.reshape(1, d)
    bias = ln_bias.reshape(1, d)

    grid = n // BLOCK_R
    s1, s2, cm1, cm2, aux = pl.pallas_call(
        functools.partial(_tc_body, grid=grid),
        grid=(grid,),
        in_specs=[
            pl.BlockSpec((BLOCK_R, d), lambda i: (i, 0)),
            pl.BlockSpec((SUB, SQRT_N), lambda i: (0, 0)),
            pl.BlockSpec((SUB, SQRT_N), lambda i: (0, 0)),
            pl.BlockSpec((1, d), lambda i: (0, 0)),
            pl.BlockSpec((1, d), lambda i: (0, 0)),
        ],
        out_specs=[
            pl.BlockSpec((4, BLOCK_R, 128), lambda i: (0, i, 0)),
            pl.BlockSpec((4, BLOCK_R, 128), lambda i: (0, i, 0)),
            pl.BlockSpec((BLOCK_R, NCHUNK), lambda i: (i, 0)),
            pl.BlockSpec((BLOCK_R, NCHUNK), lambda i: (i, 0)),
            pl.BlockSpec((1, 1), lambda i: (0, 0)),
        ],
        out_shape=[
            jax.ShapeDtypeStruct((4, n, 128), jnp.float32),
            jax.ShapeDtypeStruct((4, n, 128), jnp.float32),
            jax.ShapeDtypeStruct((n, NCHUNK), jnp.float32),
            jax.ShapeDtypeStruct((n, NCHUNK), jnp.float32),
            jax.ShapeDtypeStruct((1, 1), jnp.float32),
        ],
        scratch_shapes=[pltpu.VMEM((2, SQRT_N), jnp.float32)],
    )(q, ck1, ck2, w, bias)

    info = plsc.get_sparse_core_info()
    nw = info.num_cores * info.num_subcores
    rows_w = n // nw
    batch = SC_BATCH
    mesh = plsc.VectorSubcoreMesh(core_axis_name="c", subcore_axis_name="s")

    sc_fn = pl.kernel(
        functools.partial(_sc_body, rows_w, batch, nw),
        out_type=[
            jax.ShapeDtypeStruct((n * K,), jnp.int32),
            jax.ShapeDtypeStruct((n * K,), jnp.float32),
        ],
        mesh=mesh,
        compiler_params=pltpu.CompilerParams(use_tc_tiling_on_sc=False,
                                             needs_layout_passes=False),
        scratch_types=[
            pltpu.VMEM((batch * NCHUNK,), jnp.float32),
            pltpu.VMEM((batch * NCHUNK,), jnp.float32),
            pltpu.VMEM((batch * K + K,), jnp.int32),
            pltpu.VMEM((batch * K + K,), jnp.int32),
            pltpu.VMEM((batch * K, GSIZE), jnp.float32),
            pltpu.VMEM((batch * K, GSIZE), jnp.float32),
            pltpu.VMEM((batch * K + K,), jnp.int32),
            pltpu.VMEM((batch * K + K,), jnp.float32),
            pltpu.SemaphoreType.DMA,
            pltpu.SemaphoreType.DMA,
        ],
    )
    gidx, fsc = sc_fn(s1.reshape(n * NCHUNK, GSIZE),
                      s2.reshape(n * NCHUNK, GSIZE),
                      cm1.reshape(n * NCHUNK), cm2.reshape(n * NCHUNK))

    return (gidx.reshape(b, s, h, K), fsc.reshape(b, s, h, K),
            aux.reshape(()))


def kernel(query, c_keys, c_prime_keys, ln_weight, ln_bias):
    return _run(query, c_keys, c_prime_keys, ln_weight, ln_bias)


# speedup vs baseline: 1.7111x; 1.0038x over previous
import functools

import jax
import jax.numpy as jnp
from jax import lax
from jax.experimental import pallas as pl
from jax.experimental.pallas import tpu as pltpu
from jax.experimental.pallas import tpu_sc as plsc

D = 256
SUB = 128
SQRT_N = 512
K = 8
GSIZE = 16
NCHUNK = SQRT_N // GSIZE
LANES = 16
NEG_INF = float("-inf")

BLOCK_R = 512
SC_BATCH = 32


def _tc_body(q_ref, ck1_ref, ck2_ref, w_ref, b_ref,
             s1_ref, s2_ref, cm1_ref, cm2_ref, aux_ref, ps_ref, *, grid):
    i = pl.program_id(0)

    @pl.when(i == 0)
    def _init():
        ps_ref[...] = jnp.zeros_like(ps_ref)

    q = q_ref[...]
    mu = jnp.mean(q, axis=-1, keepdims=True)
    var = jnp.mean((q - mu) ** 2, axis=-1, keepdims=True)
    qn = (q - mu) / jnp.sqrt(var + 1e-5) * w_ref[...] + b_ref[...]

    s1 = jnp.dot(qn[:, :SUB], ck1_ref[...], preferred_element_type=jnp.float32)
    s2 = jnp.dot(qn[:, SUB:], ck2_ref[...], preferred_element_type=jnp.float32)
    for bb in range(4):
        s1_ref[bb, :, :] = s1[:, bb * 128:(bb + 1) * 128]
        s2_ref[bb, :, :] = s2[:, bb * 128:(bb + 1) * 128]

    sel = jnp.equal(
        lax.broadcasted_iota(jnp.int32, (SQRT_N, NCHUNK), 0),
        GSIZE * lax.broadcasted_iota(jnp.int32, (SQRT_N, NCHUNK), 1),
    ).astype(jnp.float32)

    def chunk_max(s):
        x = s
        for sh in (1, 2, 4, 8):
            x = jnp.maximum(x, pltpu.roll(x, SQRT_N - sh, 1))
        return jnp.dot(x, sel, preferred_element_type=jnp.float32,
                       precision=lax.Precision.HIGHEST)

    cm1_ref[...] = chunk_max(s1)
    cm2_ref[...] = chunk_max(s2)

    e1 = jnp.exp(s1)
    e2 = jnp.exp(s2)
    p1 = e1 / jnp.sum(e1, axis=-1, keepdims=True)
    p2 = e2 / jnp.sum(e2, axis=-1, keepdims=True)
    ps_ref[0:1, :] = ps_ref[0:1, :] + jnp.sum(p1, axis=0, keepdims=True)
    ps_ref[1:2, :] = ps_ref[1:2, :] + jnp.sum(p2, axis=0, keepdims=True)

    @pl.when(i == grid - 1)
    def _fin():
        n_rows = grid * BLOCK_R
        ps = ps_ref[...] * (1.0 / n_rows)
        aux_ref[...] = jnp.sum(ps * ps).reshape(1, 1) * SQRT_N


def _i16():
    return lax.iota(jnp.int32, LANES)


def _gather16(x, idx):
    return lax.gather(
        x, idx.reshape(LANES, 1),
        lax.GatherDimensionNumbers(offset_dims=(), collapsed_slice_dims=(0,),
                                   start_index_map=(0,)),
        (1,), mode=lax.GatherScatterMode.PROMISE_IN_BOUNDS)


def _merge_desc(xk, xv, yk, yv):
    ryk = lax.rev(yk, (0,))
    ryv = lax.rev(yv, (0,))
    m = xk >= ryk
    return plsc.sort_key_val(jnp.where(m, xk, ryk), jnp.where(m, xv, ryv),
                             descending=True)


def _merge_da(xk, xv, yk, yv, desc):
    m = xk >= yk
    return plsc.sort_key_val(jnp.where(m, xk, yk), jnp.where(m, xv, yv),
                             descending=desc)


def _sc_body(rows_w, batch, nw,
             s1_hbm, s2_hbm, cm1_hbm, cm2_hbm, outi_hbm, outf_hbm,
             cm1_v, cm2_v, ids1_v, ids2_v, cand1_v, cand2_v,
             outi_v, outf_v, sem1, sem2):
    wid = lax.axis_index("s") * 2 + lax.axis_index("c")
    row0 = wid * rows_w
    nb = rows_w // batch
    iota = _i16()
    n8 = rows_w * nw * 8
    shift8 = jnp.maximum(iota - K, 0)
    a8 = (iota >= 8).astype(jnp.int32)
    a12 = (iota >= 12).astype(jnp.int32)
    a14 = (iota >= 14).astype(jnp.int32)
    pi_a = a8 + a12 + a14
    pj_a = iota - (8 * a8 + 4 * a12 + 2 * a14)
    ca = pi_a * K + pj_a
    pi_b = jnp.where(iota < 4, iota + 4, 0)
    pj_b = iota * 0
    cb = pi_b * K

    def top8_chunks(cm_v, off):
        ak = cm_v[pl.ds(off, LANES)]
        bk = cm_v[pl.ds(off + LANES, LANES)]
        ask, asv = plsc.sort_key_val(ak, iota, descending=True)
        bsk, bsv = plsc.sort_key_val(bk, iota + LANES, descending=False)
        _, mv = _merge_da(ask, asv, bsk, bsv, True)
        return mv

    def batch_body(bb, carry):
        rb = row0 + bb * batch

        pltpu.sync_copy(cm1_hbm.at[pl.ds(rb * NCHUNK, batch * NCHUNK)], cm1_v)
        pltpu.sync_copy(cm2_hbm.at[pl.ds(rb * NCHUNK, batch * NCHUNK)], cm2_v)

        def pair_body(pp, c):
            r = 2 * pp
            for cm_v, ids_v, cand_v, s_hbm, sem in (
                (cm1_v, ids1_v, cand1_v, s1_hbm, sem1),
                (cm2_v, ids2_v, cand2_v, s2_hbm, sem2),
            ):
                mv0 = top8_chunks(cm_v, r * NCHUNK)
                mv1 = top8_chunks(cm_v, (r + 1) * NCHUNK)
                ids16 = jnp.where(iota < K, mv0, _gather16(mv1, shift8))
                ids_v[pl.ds(r * K, LANES)] = ids16
                fidx = (lax.shift_right_logical(ids16, 3) * n8
                        + ((rb + r) + a8) * 8 + jnp.bitwise_and(ids16, 7))
                pltpu.async_copy(s_hbm.at[fidx],
                                 cand_v.at[pl.ds(r * K, LANES)], sem)
            return c
        lax.fori_loop(0, batch // 2, pair_body, 0)

        pltpu.make_async_copy(s1_hbm.at[pl.ds(0, batch * K)], cand1_v, sem1).wait()
        pltpu.make_async_copy(s2_hbm.at[pl.ds(0, batch * K)], cand2_v, sem2).wait()

        def row_body(r, c):
            tops = []
            for ids_v, cand_v in ((ids1_v, cand1_v), (ids2_v, cand2_v)):
                idsvec = ids_v[pl.ds(r * K, LANES)]
                runs = []
                for j in range(K):
                    p = r * K + j
                    v = cand_v[p, :]
                    runs.append(plsc.sort_key_val(v, iota + j * GSIZE,
                                                  descending=(j % 2 == 0)))
                while len(runs) > 1:
                    runs = [_merge_da(*runs[t], *runs[t + 1],
                                      (t // 2) % 2 == 0 or len(runs) == 2)
                            for t in range(0, len(runs), 2)]
                sk, sp = runs[0]
                sv = (_gather16(idsvec, lax.shift_right_logical(sp, 4)) * GSIZE
                      + jnp.bitwise_and(sp, GSIZE - 1))
                tops.append((sk, sv))
            (s1k, s1v), (s2k, s2v) = tops

            ka = _gather16(s1k, pi_a) + _gather16(s2k, pj_a)
            kb = _gather16(s1k, pi_b) + _gather16(s2k, pj_b)
            kb = jnp.where(iota < 4, kb, NEG_INF)
            ask, acs = plsc.sort_key_val(ka, ca, descending=True)
            fk, fc = _merge_desc(ask, acs, kb, cb)

            fi = lax.shift_right_logical(fc, 3)
            fj = jnp.bitwise_and(fc, 7)
            rr = _gather16(s1v, fi)
            rc = _gather16(s2v, fj)
            gi = rr * SQRT_N + rc
            outi_v[pl.ds(r * K, LANES)] = gi
            outf_v[pl.ds(r * K, LANES)] = fk
            return c
        lax.fori_loop(0, batch, row_body, 0)

        pltpu.sync_copy(outi_v.at[pl.ds(0, batch * K)],
                        outi_hbm.at[pl.ds(rb * K, batch * K)])
        pltpu.sync_copy(outf_v.at[pl.ds(0, batch * K)],
                        outf_hbm.at[pl.ds(rb * K, batch * K)])
        return carry

    lax.fori_loop(0, nb, batch_body, 0)


@jax.jit
def _run(query, c_keys, c_prime_keys, ln_weight, ln_bias):
    b, s, h, d = query.shape
    n = b * s * h
    q = query.reshape(n, d)
    ck1 = c_keys.T
    ck2 = c_prime_keys.T
    w = ln_weight.reshape(1, d)
    bias = ln_bias.reshape(1, d)

    grid = n // BLOCK_R
    s1, s2, cm1, cm2, aux = pl.pallas_call(
        functools.partial(_tc_body, grid=grid),
        grid=(grid,),
        in_specs=[
            pl.BlockSpec((BLOCK_R, d), lambda i: (i, 0)),
            pl.BlockSpec((SUB, SQRT_N), lambda i: (0, 0)),
            pl.BlockSpec((SUB, SQRT_N), lambda i: (0, 0)),
            pl.BlockSpec((1, d), lambda i: (0, 0)),
            pl.BlockSpec((1, d), lambda i: (0, 0)),
        ],
        out_specs=[
            pl.BlockSpec((4, BLOCK_R, 128), lambda i: (0, i, 0)),
            pl.BlockSpec((4, BLOCK_R, 128), lambda i: (0, i, 0)),
            pl.BlockSpec((BLOCK_R, NCHUNK), lambda i: (i, 0)),
            pl.BlockSpec((BLOCK_R, NCHUNK), lambda i: (i, 0)),
            pl.BlockSpec((1, 1), lambda i: (0, 0)),
        ],
        out_shape=[
            jax.ShapeDtypeStruct((4, n, 128), jnp.float32),
            jax.ShapeDtypeStruct((4, n, 128), jnp.float32),
            jax.ShapeDtypeStruct((n, NCHUNK), jnp.float32),
            jax.ShapeDtypeStruct((n, NCHUNK), jnp.float32),
            jax.ShapeDtypeStruct((1, 1), jnp.float32),
        ],
        scratch_shapes=[pltpu.VMEM((2, SQRT_N), jnp.float32)],
    )(q, ck1, ck2, w, bias)

    info = plsc.get_sparse_core_info()
    nw = info.num_cores * info.num_subcores
    rows_w = n // nw
    batch = SC_BATCH
    mesh = plsc.VectorSubcoreMesh(core_axis_name="c", subcore_axis_name="s")

    sc_fn = pl.kernel(
        functools.partial(_sc_body, rows_w, batch, nw),
        out_type=[
            jax.ShapeDtypeStruct((n * K,), jnp.int32),
            jax.ShapeDtypeStruct((n * K,), jnp.float32),
        ],
        mesh=mesh,
        compiler_params=pltpu.CompilerParams(use_tc_tiling_on_sc=False,
                                             needs_layout_passes=False),
        scratch_types=[
            pltpu.VMEM((batch * NCHUNK,), jnp.float32),
            pltpu.VMEM((batch * NCHUNK,), jnp.float32),
            pltpu.VMEM((batch * K + K,), jnp.int32),
            pltpu.VMEM((batch * K + K,), jnp.int32),
            pltpu.VMEM((batch * K, GSIZE), jnp.float32),
            pltpu.VMEM((batch * K, GSIZE), jnp.float32),
            pltpu.VMEM((batch * K + K,), jnp.int32),
            pltpu.VMEM((batch * K + K,), jnp.float32),
            pltpu.SemaphoreType.DMA,
            pltpu.SemaphoreType.DMA,
        ],
    )
    gidx, fsc = sc_fn(s1.reshape(n * NCHUNK, GSIZE),
                      s2.reshape(n * NCHUNK, GSIZE),
                      cm1.reshape(n * NCHUNK), cm2.reshape(n * NCHUNK))

    return (gidx.reshape(b, s, h, K), fsc.reshape(b, s, h, K),
            aux.reshape(()))


def kernel(query, c_keys, c_prime_keys, ln_weight, ln_bias):
    return _run(query, c_keys, c_prime_keys, ln_weight, ln_bias)
